# baseline (speedup 1.0000x reference)
import functools

import jax
import jax.numpy as jnp
from jax import lax
from jax.experimental import pallas as pl
from jax.experimental.pallas import tpu as pltpu
from jax.experimental.pallas import tpu_sc as plsc

_NC = 2
_NS = 16
_NW = _NC * _NS

_CHUNK = 640
_NBUF = 4


@functools.partial(jax.jit, static_argnums=(2, 3))
def _sc_gather(idx, table, B, D):
    bpw = B // _NW
    n_chunks = bpw // _CHUNK
    mesh = plsc.VectorSubcoreMesh(core_axis_name="c", subcore_axis_name="s")

    @functools.partial(
        pl.kernel,
        mesh=mesh,
        out_type=jax.ShapeDtypeStruct((B, D), jnp.float32),
        scratch_types=[
            pltpu.VMEM((bpw,), jnp.int32),
        ] + [pltpu.VMEM((_CHUNK, D), jnp.float32) for _ in range(_NBUF)] + [
            pltpu.SemaphoreType.DMA((_NBUF,)),
            pltpu.SemaphoreType.DMA((_NBUF,)),
        ],
        compiler_params=pltpu.CompilerParams(use_tc_tiling_on_sc=False),
    )
    def k(idx_hbm, table_hbm, out_hbm, idx_v, *rest):
        bufs = rest[:_NBUF]
        gsem, wsem = rest[_NBUF], rest[_NBUF + 1]
        wid = lax.axis_index("s") * _NC + lax.axis_index("c")
        base = wid * bpw

        pltpu.sync_copy(idx_hbm.at[pl.ds(base, bpw)], idx_v)

        def gather_start(i, b):
            pltpu.make_async_copy(
                table_hbm.at[idx_v.at[pl.ds(i * _CHUNK, _CHUNK)]],
                bufs[b], gsem.at[b]).start()

        def gather_wait(b):
            pltpu.make_async_copy(
                table_hbm.at[idx_v.at[pl.ds(0, _CHUNK)]],
                bufs[b], gsem.at[b]).wait()

        def wb_start(i, b):
            pltpu.make_async_copy(
                bufs[b], out_hbm.at[pl.ds(base + i * _CHUNK, _CHUNK)],
                wsem.at[b]).start()

        def wb_wait(b):
            pltpu.make_async_copy(
                bufs[b], out_hbm.at[pl.ds(base, _CHUNK)],
                wsem.at[b]).wait()

        for i in range(_NBUF - 1):
            gather_start(i, i % _NBUF)
        for i in range(n_chunks):
            b = i % _NBUF
            if i + _NBUF - 1 < n_chunks:
                nb = (i + _NBUF - 1) % _NBUF
                if i >= 1:
                    wb_wait(nb)
                gather_start(i + _NBUF - 1, nb)
            gather_wait(b)
            wb_start(i, b)
        for j in range(min(_NBUF, n_chunks)):
            wb_wait((n_chunks - 1 - j) % _NBUF)

    return k(idx, table)


def kernel(x, table):
    D = table.shape[1]
    idx = x.reshape(-1).astype(jnp.int32)
    B = idx.shape[0]
    out = _sc_gather(idx, table, B, D)
    return out.reshape(x.shape + (D,))
